# Initial kernel scaffold; baseline (speedup 1.0000x reference)
import jax
import jax.numpy as jnp
from jax.experimental import pallas as pl


def kernel(h_src, h_dst, edge_index):
    raise NotImplementedError("write your pallas kernel here")


# speedup vs baseline: 4.6745x; 4.6745x over previous
import dataclasses
import functools

import jax
import jax.numpy as jnp
from jax import lax
from jax.experimental import pallas as pl
from jax.experimental.pallas import tpu as pltpu
from jax.experimental.pallas import tpu_sc as plsc

_NC = 2
_NS = 16
_L = 16
_W = 128


@functools.lru_cache(maxsize=None)
def _build(E, D):
    assert E % _W == 0
    n_chunks = E // _W
    nw = _NC * _NS
    per_w = (n_chunks + nw - 1) // nw
    mesh = plsc.VectorSubcoreMesh(core_axis_name="c", subcore_axis_name="s")
    cp = pltpu.CompilerParams()
    if "needs_layout_passes" in pltpu.CompilerParams.__dataclass_fields__:
        cp = dataclasses.replace(cp, needs_layout_passes=False)

    @functools.partial(
        pl.kernel,
        compiler_params=cp,
        out_type=jax.ShapeDtypeStruct((E,), jnp.float32),
        mesh=mesh,
        scratch_types=[
            pltpu.VMEM((_W,), jnp.int32),
            pltpu.VMEM((_W,), jnp.int32),
            pltpu.VMEM((_W, D), jnp.float32),
            pltpu.VMEM((_W, D), jnp.float32),
            pltpu.VMEM((_W,), jnp.float32),
            pltpu.SemaphoreType.DMA,
            pltpu.SemaphoreType.DMA,
        ],
    )
    def edge_dot(hsrc, hdst, sidx, didx, out, sidx_v, didx_v, u_v, v_v,
                 out_v, sem1, sem2):
        wid = lax.axis_index("s") * _NC + lax.axis_index("c")
        lane = lax.iota(jnp.int32, _L)

        @pl.loop(0, per_w)
        def _(i):
            chunk = wid + i * nw

            @pl.when(chunk < n_chunks)
            def _():
                base = chunk * _W
                pltpu.sync_copy(sidx.at[pl.ds(base, _W)], sidx_v)
                pltpu.sync_copy(didx.at[pl.ds(base, _W)], didx_v)
                cp1 = pltpu.async_copy(hsrc.at[sidx_v], u_v, sem1)
                cp2 = pltpu.async_copy(hdst.at[didx_v], v_v, sem2)
                cp1.wait()
                cp2.wait()

                @pl.loop(0, _W // _L)
                def _(g):
                    def row(j, res):
                        r = g * _L + j
                        acc = u_v[r, pl.ds(0, _L)] * v_v[r, pl.ds(0, _L)]
                        for c in range(1, D // _L):
                            acc = acc + (u_v[r, pl.ds(c * _L, _L)]
                                         * v_v[r, pl.ds(c * _L, _L)])
                        s = jnp.sum(acc)
                        return jnp.where(lane == j, s, res)

                    res = lax.fori_loop(0, _L, row,
                                        jnp.zeros((_L,), jnp.float32))
                    out_v[pl.ds(g * _L, _L)] = res

                pltpu.sync_copy(out_v, out.at[pl.ds(base, _W)])

    return edge_dot


def kernel(h_src, h_dst, edge_index):
    idx = edge_index.astype(jnp.int32)
    fn = _build(edge_index.shape[1], h_src.shape[1])
    return fn(h_src, h_dst, idx[0], idx[1])


# speedup vs baseline: 9.1695x; 1.9616x over previous
import dataclasses
import functools

import jax
import jax.numpy as jnp
from jax import lax
from jax.experimental import pallas as pl
from jax.experimental.pallas import tpu as pltpu
from jax.experimental.pallas import tpu_sc as plsc

_NC = 2
_NS = 16
_L = 16
_LB = 32
_W = 128


@functools.lru_cache(maxsize=None)
def _build(E, D):
    nw = _NC * _NS
    assert E % (nw * _L) == 0 and D % _LB == 0
    Dw = D // 2
    n_e = E // nw
    n_full = n_e // _W
    n_tail = n_e - n_full * _W
    assert n_tail % _L == 0

    mesh = plsc.VectorSubcoreMesh(core_axis_name="c", subcore_axis_name="s")
    cp = pltpu.CompilerParams()
    if "needs_layout_passes" in pltpu.CompilerParams.__dataclass_fields__:
        cp = dataclasses.replace(cp, needs_layout_passes=False)
    if "use_tc_tiling_on_sc" in pltpu.CompilerParams.__dataclass_fields__:
        cp = dataclasses.replace(cp, use_tc_tiling_on_sc=False)

    @functools.partial(
        pl.kernel,
        compiler_params=cp,
        out_type=jax.ShapeDtypeStruct((E,), jnp.float32),
        mesh=mesh,
        scratch_types=[
            pltpu.VMEM((n_e,), jnp.int32),
            pltpu.VMEM((n_e,), jnp.int32),
            pltpu.VMEM((2, _W, Dw), jnp.int32),
            pltpu.VMEM((2, _W, Dw), jnp.int32),
            pltpu.VMEM((max(n_tail, 1), Dw), jnp.int32),
            pltpu.VMEM((max(n_tail, 1), Dw), jnp.int32),
            pltpu.VMEM((n_e,), jnp.float32),
            pltpu.SemaphoreType.DMA,
            pltpu.SemaphoreType.DMA,
            pltpu.SemaphoreType.DMA,
        ],
    )
    def edge_dot(hsrc, hdst, sidx, didx, out, sidx_v, didx_v, u_v, v_v,
                 ut_v, vt_v, out_v, sem0, sem1, semt):
        wid = lax.axis_index("s") * _NC + lax.axis_index("c")
        span = wid * n_e
        lane = lax.iota(jnp.int32, _L)
        sems = (sem0, sem1)

        pltpu.sync_copy(sidx.at[pl.ds(span, n_e)], sidx_v)
        pltpu.sync_copy(didx.at[pl.ds(span, n_e)], didx_v)

        def fire(c, b):
            pltpu.async_copy(hsrc.at[sidx_v.at[pl.ds(c * _W, _W)]],
                             u_v.at[b], sems[b])
            pltpu.async_copy(hdst.at[didx_v.at[pl.ds(c * _W, _W)]],
                             v_v.at[b], sems[b])

        def drain(b):
            pltpu.make_async_copy(hsrc.at[sidx_v.at[pl.ds(0, _W)]],
                                  u_v.at[b], sems[b]).wait()
            pltpu.make_async_copy(hdst.at[didx_v.at[pl.ds(0, _W)]],
                                  v_v.at[b], sems[b]).wait()

        def rows16(u_ref, v_ref, g, obase):
            res = jnp.zeros((_L,), jnp.float32)
            for j in range(_L):
                r = g * _L + j
                acc = None
                for c in range(D // _LB):
                    lu = plsc.bitcast(u_ref[r, pl.ds(c * _L, _L)],
                                      jnp.bfloat16)
                    lv = plsc.bitcast(v_ref[r, pl.ds(c * _L, _L)],
                                      jnp.bfloat16)
                    p = lu * lv
                    pa, pb = plsc.unpack(p, format=plsc.PackFormat.INTERLEAVED)
                    s = pa + pb
                    acc = s if acc is None else acc + s
                res = jnp.where(lane == j, jnp.sum(acc), res)
            out_v[pl.ds(obase + g * _L, _L)] = res

        if n_tail:
            pltpu.async_copy(
                hsrc.at[sidx_v.at[pl.ds(n_full * _W, n_tail)]], ut_v, semt)
            pltpu.async_copy(
                hdst.at[didx_v.at[pl.ds(n_full * _W, n_tail)]], vt_v, semt)
        fire(0, 0)

        def chunk_body(c, b):
            @pl.when(c + 1 < n_full)
            def _():
                fire(c + 1, 1 - b)
            drain(b)

            @pl.loop(0, _W // _L)
            def _(g):
                rows16(u_v.at[b], v_v.at[b], g, c * _W)

        @pl.loop(0, n_full // 2)
        def _(i):
            chunk_body(2 * i, 0)
            chunk_body(2 * i + 1, 1)
        if n_full % 2:
            chunk_body(n_full - 1, 0)

        if n_tail:
            pltpu.make_async_copy(
                hsrc.at[sidx_v.at[pl.ds(0, n_tail)]], ut_v, semt).wait()
            pltpu.make_async_copy(
                hdst.at[didx_v.at[pl.ds(0, n_tail)]], vt_v, semt).wait()
            for g in range(n_tail // _L):
                rows16(ut_v, vt_v, g, n_full * _W)

        pltpu.sync_copy(out_v, out.at[pl.ds(span, n_e)])

    return edge_dot


def _as_i32_rows(h):
    n, d = h.shape
    hb = h.astype(jnp.bfloat16).reshape(n, d // 2, 2)
    return jax.lax.bitcast_convert_type(hb, jnp.int32)


def kernel(h_src, h_dst, edge_index):
    idx = edge_index.astype(jnp.int32)
    fn = _build(edge_index.shape[1], h_src.shape[1])
    return fn(_as_i32_rows(h_src), _as_i32_rows(h_dst), idx[0], idx[1])


# speedup vs baseline: 12.6504x; 1.3796x over previous
import dataclasses
import functools

import jax
import jax.numpy as jnp
from jax import lax
from jax.experimental import pallas as pl
from jax.experimental.pallas import tpu as pltpu
from jax.experimental.pallas import tpu_sc as plsc

_NC = 2
_NS = 16
_L = 16
_LB = 32
_W = 128
_CVT_UNROLL = 5


def _compiler_params():
    cp = pltpu.CompilerParams()
    if "needs_layout_passes" in pltpu.CompilerParams.__dataclass_fields__:
        cp = dataclasses.replace(cp, needs_layout_passes=False)
    if "use_tc_tiling_on_sc" in pltpu.CompilerParams.__dataclass_fields__:
        cp = dataclasses.replace(cp, use_tc_tiling_on_sc=False)
    return cp


@functools.lru_cache(maxsize=None)
def _build_convert(n_words):
    nw = _NC * _NS
    assert n_words % (nw * _L * _CVT_UNROLL) == 0
    per_w = n_words // nw
    n_iter = per_w // (_L * _CVT_UNROLL)
    mesh = plsc.VectorSubcoreMesh(core_axis_name="c", subcore_axis_name="s")

    @functools.partial(
        pl.kernel,
        compiler_params=_compiler_params(),
        out_type=jax.ShapeDtypeStruct((n_words,), jnp.int32),
        mesh=mesh,
        scratch_types=[
            pltpu.VMEM((2 * per_w,), jnp.float32),
            pltpu.VMEM((per_w,), jnp.int32),
        ],
    )
    def convert(src, out, in_v, out_v):
        wid = lax.axis_index("s") * _NC + lax.axis_index("c")
        pltpu.sync_copy(src.at[pl.ds(wid * 2 * per_w, 2 * per_w)], in_v)

        @pl.loop(0, n_iter)
        def _(i):
            for u in range(_CVT_UNROLL):
                base = (i * _CVT_UNROLL + u) * _LB
                a = in_v[pl.ds(base, _L)]
                b = in_v[pl.ds(base + _L, _L)]
                w = plsc.bitcast(
                    plsc.pack(a, b, format=plsc.PackFormat.INTERLEAVED),
                    jnp.int32)
                out_v[pl.ds((i * _CVT_UNROLL + u) * _L, _L)] = w

        pltpu.sync_copy(out_v, out.at[pl.ds(wid * per_w, per_w)])

    return convert


@functools.lru_cache(maxsize=None)
def _build_main(E, D):
    nw = _NC * _NS
    assert E % (nw * _L) == 0 and D % _LB == 0
    Dw = D // 2
    n_e = E // nw
    n_full = n_e // _W
    n_tail = n_e - n_full * _W
    assert n_tail % _L == 0
    mesh = plsc.VectorSubcoreMesh(core_axis_name="c", subcore_axis_name="s")

    @functools.partial(
        pl.kernel,
        compiler_params=_compiler_params(),
        out_type=jax.ShapeDtypeStruct((E,), jnp.float32),
        mesh=mesh,
        scratch_types=[
            pltpu.VMEM((n_e,), jnp.int32),
            pltpu.VMEM((n_e,), jnp.int32),
            pltpu.VMEM((2, _W, Dw), jnp.int32),
            pltpu.VMEM((2, _W, Dw), jnp.int32),
            pltpu.VMEM((max(n_tail, 1), Dw), jnp.int32),
            pltpu.VMEM((max(n_tail, 1), Dw), jnp.int32),
            pltpu.VMEM((n_e,), jnp.float32),
            pltpu.SemaphoreType.DMA,
            pltpu.SemaphoreType.DMA,
            pltpu.SemaphoreType.DMA,
        ],
    )
    def edge_dot(hsrc, hdst, ei, out, sidx_v, didx_v, u_v, v_v,
                 ut_v, vt_v, out_v, sem0, sem1, semt):
        wid = lax.axis_index("s") * _NC + lax.axis_index("c")
        span = wid * n_e
        lane = lax.iota(jnp.int32, _L)
        sems = (sem0, sem1)

        pltpu.sync_copy(ei.at[0, pl.ds(span, n_e)], sidx_v)
        pltpu.sync_copy(ei.at[1, pl.ds(span, n_e)], didx_v)

        def fire(c, b):
            pltpu.async_copy(hsrc.at[sidx_v.at[pl.ds(c * _W, _W)]],
                             u_v.at[b], sems[b])
            pltpu.async_copy(hdst.at[didx_v.at[pl.ds(c * _W, _W)]],
                             v_v.at[b], sems[b])

        def drain(b):
            pltpu.make_async_copy(hsrc.at[sidx_v.at[pl.ds(0, _W)]],
                                  u_v.at[b], sems[b]).wait()
            pltpu.make_async_copy(hdst.at[didx_v.at[pl.ds(0, _W)]],
                                  v_v.at[b], sems[b]).wait()

        def rows16(u_ref, v_ref, g, obase):
            res = jnp.zeros((_L,), jnp.float32)
            for j in range(_L):
                r = g * _L + j
                ps = []
                for c in range(D // _LB):
                    lu = plsc.bitcast(u_ref[r, pl.ds(c * _L, _L)],
                                      jnp.bfloat16)
                    lv = plsc.bitcast(v_ref[r, pl.ds(c * _L, _L)],
                                      jnp.bfloat16)
                    ps.append(lu * lv)
                while len(ps) > 1:
                    ps = [a + b for a, b in zip(ps[::2], ps[1::2])]
                pa, pb = plsc.unpack(ps[0], format=plsc.PackFormat.INTERLEAVED)
                res = jnp.where(lane == j, jnp.sum(pa + pb), res)
            out_v[pl.ds(obase + g * _L, _L)] = res

        if n_tail:
            pltpu.async_copy(
                hsrc.at[sidx_v.at[pl.ds(n_full * _W, n_tail)]], ut_v, semt)
            pltpu.async_copy(
                hdst.at[didx_v.at[pl.ds(n_full * _W, n_tail)]], vt_v, semt)
        fire(0, 0)

        def chunk_body(c, b):
            @pl.when(c + 1 < n_full)
            def _():
                fire(c + 1, 1 - b)
            drain(b)

            @pl.loop(0, _W // _L)
            def _(g):
                rows16(u_v.at[b], v_v.at[b], g, c * _W)

        @pl.loop(0, n_full // 2)
        def _(i):
            chunk_body(2 * i, 0)
            chunk_body(2 * i + 1, 1)
        if n_full % 2:
            chunk_body(n_full - 1, 0)

        if n_tail:
            pltpu.make_async_copy(
                hsrc.at[sidx_v.at[pl.ds(0, n_tail)]], ut_v, semt).wait()
            pltpu.make_async_copy(
                hdst.at[didx_v.at[pl.ds(0, n_tail)]], vt_v, semt).wait()
            for g in range(n_tail // _L):
                rows16(ut_v, vt_v, g, n_full * _W)

        pltpu.sync_copy(out_v, out.at[pl.ds(span, n_e)])

    return edge_dot


def kernel(h_src, h_dst, edge_index):
    n, d = h_src.shape
    cvt = _build_convert(n * d // 2)
    hsrc_p = cvt(h_src.reshape(-1)).reshape(n, d // 2)
    hdst_p = cvt(h_dst.reshape(-1)).reshape(n, d // 2)
    fn = _build_main(edge_index.shape[1], d)
    return fn(hsrc_p, hdst_p, edge_index.astype(jnp.int32))
